# baseline (speedup 1.0000x reference)
import functools
import math

import jax
import jax.numpy as jnp
import numpy as np
from jax import lax
from jax.experimental import pallas as pl
from jax.experimental.pallas import tpu as pltpu
from jax.experimental.pallas import tpu_sc as plsc

_B, _C, _H, _W = 8, 96, 56, 56
_K = 9
_HR, _WR = 7, 7
_N = _H * _W
_NR = _HR * _WR
_GROUPS = 4
_BIG = 3.0e38

_NC, _NS, _L = 2, 16, 16
_NW = _NC * _NS
_NODES_PER_W = (_B * _N) // _NW


def _pos_embed_np(c, h, w):
    d = c // 2
    pe = np.zeros((c, h, w), dtype=np.float32)
    div = np.exp(np.arange(0.0, d, 2) * -(math.log(10000.0) / d))
    pos_w = np.arange(0.0, w)[:, None]
    pos_h = np.arange(0.0, h)[:, None]
    pe[0:d:2, :, :] = np.sin(pos_w * div).T[:, None, :]
    pe[1:d:2, :, :] = np.cos(pos_w * div).T[:, None, :]
    pe[d::2, :, :] = np.sin(pos_h * div).T[:, :, None]
    pe[d + 1::2, :, :] = np.cos(pos_h * div).T[:, :, None]
    return pe


def _constants():
    pos = _pos_embed_np(_C, _H, _W)
    pos_red = pos.reshape(_C, _HR, _H // _HR, _WR, _W // _WR).mean(axis=(2, 4))
    rel = 2.0 * (pos.reshape(_C, -1).T @ pos_red.reshape(_C, -1)) / _C
    relT = np.ascontiguousarray(rel.T).astype(np.float32)
    hh = np.arange(_H)[:, None]
    ww = np.arange(_W)[None, :]
    blk = (hh // (_H // _HR)) * _WR + (ww // (_W // _WR))
    pool = (blk.reshape(_N, 1) == np.arange(_NR)[None, :]).astype(np.float32) / 64.0
    return relT, pool


_RELT_NP, _POOL_NP = _constants()


def _tc1_body(x_ref, w1_ref, b1_ref, relT_ref, pool_ref,
              h_ref, yT_ref, idx_ref):
    x = x_ref[0]
    h = jnp.dot(w1_ref[...], x, preferred_element_type=jnp.float32) + b1_ref[...]
    h_ref[0] = h

    y = jnp.dot(h, pool_ref[...], preferred_element_type=jnp.float32)
    yT_ref[0] = jax.lax.dot_general(pool_ref[...], h, (((0,), (1,)), ((), ())),
                                    preferred_element_type=jnp.float32)

    nx = jnp.sqrt(jnp.sum(h * h, axis=0, keepdims=True))
    ny = jnp.sqrt(jnp.sum(y * y, axis=0, keepdims=True))
    ipT = jax.lax.dot_general(y, h, (((0,), (0,)), ((), ())),
                              preferred_element_type=jnp.float32)
    inv_x = 1.0 / (nx + 1e-12)
    inv_y = 1.0 / (ny + 1e-12)
    innerT = ipT * inv_x * inv_y.reshape(_NR, 1)
    sx = (nx * inv_x) ** 2
    sy = (ny * inv_y) ** 2
    distT = 2.0 * innerT - sx - sy.reshape(_NR, 1) + relT_ref[...]

    iota0 = jax.lax.broadcasted_iota(jnp.int32, (_NR, _N), 0)
    d = distT
    rows = []
    for _ in range(_K):
        cur = jnp.max(d, axis=0, keepdims=True)
        first = jnp.min(jnp.where(d >= cur, iota0, _NR), axis=0,
                        keepdims=True)
        rows.append(first)
        d = jnp.where(iota0 == first, -_BIG, d)
    idxmat = jnp.concatenate(rows, axis=0)
    for q in range(_N // _NODES_PER_W):
        idx_ref[0, q] = idxmat[:, q * _NODES_PER_W:(q + 1) * _NODES_PER_W]


def _sc_body(yT_hbm, idx_hbm, out_hbm, yT_v, idx_v, acc_v):
    wid = lax.axis_index("s") * _NC + lax.axis_index("c")
    b = wid // (_N // _NODES_PER_W)
    q = wid % (_N // _NODES_PER_W)

    pltpu.sync_copy(yT_hbm.at[b], yT_v)
    pltpu.sync_copy(idx_hbm.at[b, q], idx_v)

    def grp_body(g, carry):
        base = g * _L
        ivs = [idx_v[k, pl.ds(base, _L)] for k in range(_K)]

        def c_body(c, carry2):
            csplat = jnp.full((_L,), 0, jnp.int32) + c
            o = plsc.load_gather(yT_v, [ivs[0], csplat])
            for k in range(1, _K):
                o = jnp.maximum(o, plsc.load_gather(yT_v, [ivs[k], csplat]))
            acc_v[c, pl.ds(base, _L)] = o
            return carry2

        return lax.fori_loop(0, _C, c_body, carry)

    lax.fori_loop(0, _NODES_PER_W // _L, grp_body, 0)
    pltpu.sync_copy(acc_v, out_hbm.at[b, q])


def _tc2_body(h_ref, acc_ref, x_ref, mrw_ref, mrb_ref, mrg_ref, mrbeta_ref,
              w2_ref, b2_ref, out_ref):
    h = h_ref[0]
    acc = jnp.concatenate(
        [acc_ref[0, q] for q in range(_N // _NODES_PER_W)], axis=1)
    cat = jnp.concatenate([h, acc - h], axis=0)
    g = jnp.dot(mrw_ref[...], cat,
                preferred_element_type=jnp.float32) + mrb_ref[...]

    rows = (2 * _C) // _GROUPS
    parts = []
    for gi in range(_GROUPS):
        sub = g[gi * rows:(gi + 1) * rows, :]
        m = jnp.mean(sub)
        dsub = sub - m
        v = jnp.mean(dsub * dsub)
        parts.append(dsub * jax.lax.rsqrt(v + 1e-5))
    gn = jnp.concatenate(parts, axis=0) * mrg_ref[...] + mrbeta_ref[...]
    act = jax.nn.gelu(gn)

    out_ref[0] = (jnp.dot(w2_ref[...], act, preferred_element_type=jnp.float32)
                  + b2_ref[...] + x_ref[0])


def kernel(x, fc1_w, fc1_b, fc1_g, fc1_beta, mr_w, mr_b, mr_g, mr_beta,
           fc2_w, fc2_b, fc2_g, fc2_beta):
    x3 = x.reshape(_B, _C, _N)
    w1 = fc1_g[:, None] * fc1_w
    b1 = (fc1_g * fc1_b + fc1_beta)[:, None]
    w2 = fc2_g[:, None] * fc2_w
    b2 = (fc2_g * fc2_b + fc2_beta)[:, None]
    relT = jnp.asarray(_RELT_NP)
    pool = jnp.asarray(_POOL_NP)

    full = lambda shape: pl.BlockSpec(shape, lambda b: (0,) * len(shape))
    batch = lambda shape: pl.BlockSpec((1,) + shape,
                                       lambda b: (b,) + (0,) * len(shape))

    h, yT, idx = pl.pallas_call(
        _tc1_body,
        grid=(_B,),
        in_specs=[
            batch((_C, _N)),
            full((_C, _C)), full((_C, 1)),
            full((_NR, _N)), full((_N, _NR)),
        ],
        out_specs=[batch((_C, _N)), batch((_NR, _C)),
                   batch((_N // _NODES_PER_W, _K, _NODES_PER_W))],
        out_shape=[
            jax.ShapeDtypeStruct((_B, _C, _N), jnp.float32),
            jax.ShapeDtypeStruct((_B, _NR, _C), jnp.float32),
            jax.ShapeDtypeStruct(
                (_B, _N // _NODES_PER_W, _K, _NODES_PER_W), jnp.int32),
        ],
    )(x3, w1, b1, relT, pool)

    mesh = plsc.VectorSubcoreMesh(core_axis_name="c", subcore_axis_name="s")
    acc = pl.kernel(
        _sc_body, mesh=mesh,
        compiler_params=pltpu.CompilerParams(needs_layout_passes=False),
        out_type=jax.ShapeDtypeStruct(
            (_B, _N // _NODES_PER_W, _C, _NODES_PER_W), jnp.float32),
        scratch_types=[
            pltpu.VMEM((_NR, _C), jnp.float32),
            pltpu.VMEM((_K, _NODES_PER_W), jnp.int32),
            pltpu.VMEM((_C, _NODES_PER_W), jnp.float32),
        ],
    )(yT, idx)

    out = pl.pallas_call(
        _tc2_body,
        grid=(_B,),
        in_specs=[
            batch((_C, _N)),
            batch((_N // _NODES_PER_W, _C, _NODES_PER_W)),
            batch((_C, _N)),
            full((2 * _C, 2 * _C)), full((2 * _C, 1)),
            full((2 * _C, 1)), full((2 * _C, 1)),
            full((_C, 2 * _C)), full((_C, 1)),
        ],
        out_specs=batch((_C, _N)),
        out_shape=jax.ShapeDtypeStruct((_B, _C, _N), jnp.float32),
    )(h, acc, x3, mr_w, mr_b[:, None], mr_g[:, None], mr_beta[:, None],
      w2, b2)
    return out.reshape(_B, _C, _H, _W)


# speedup vs baseline: 1.0023x; 1.0023x over previous
import functools
import math

import jax
import jax.numpy as jnp
import numpy as np
from jax import lax
from jax.experimental import pallas as pl
from jax.experimental.pallas import tpu as pltpu
from jax.experimental.pallas import tpu_sc as plsc

_B, _C, _H, _W = 8, 96, 56, 56
_K = 9
_HR, _WR = 7, 7
_N = _H * _W
_NR = _HR * _WR
_GROUPS = 4
_BIG = 3.0e38

_NC, _NS, _L = 2, 16, 16
_NW = _NC * _NS
_NODES_PER_W = (_B * _N) // _NW


def _pos_embed_np(c, h, w):
    d = c // 2
    pe = np.zeros((c, h, w), dtype=np.float32)
    div = np.exp(np.arange(0.0, d, 2) * -(math.log(10000.0) / d))
    pos_w = np.arange(0.0, w)[:, None]
    pos_h = np.arange(0.0, h)[:, None]
    pe[0:d:2, :, :] = np.sin(pos_w * div).T[:, None, :]
    pe[1:d:2, :, :] = np.cos(pos_w * div).T[:, None, :]
    pe[d::2, :, :] = np.sin(pos_h * div).T[:, :, None]
    pe[d + 1::2, :, :] = np.cos(pos_h * div).T[:, :, None]
    return pe


def _constants():
    pos = _pos_embed_np(_C, _H, _W)
    pos_red = pos.reshape(_C, _HR, _H // _HR, _WR, _W // _WR).mean(axis=(2, 4))
    rel = 2.0 * (pos.reshape(_C, -1).T @ pos_red.reshape(_C, -1)) / _C
    relT = np.ascontiguousarray(rel.T).astype(np.float32)
    hh = np.arange(_H)[:, None]
    ww = np.arange(_W)[None, :]
    blk = (hh // (_H // _HR)) * _WR + (ww // (_W // _WR))
    pool = (blk.reshape(_N, 1) == np.arange(_NR)[None, :]).astype(np.float32) / 64.0
    return relT, pool


_RELT_NP, _POOL_NP = _constants()


def _tc1_body(x_ref, w1_ref, b1_ref, relT_ref, pool_ref,
              h_ref, yT_ref, idx_ref):
    x = x_ref[0]
    h = jnp.dot(w1_ref[...], x, preferred_element_type=jnp.float32) + b1_ref[...]
    h_ref[0] = h

    y = jnp.dot(h, pool_ref[...], preferred_element_type=jnp.float32)
    yT_ref[0] = jax.lax.dot_general(pool_ref[...], h, (((0,), (1,)), ((), ())),
                                    preferred_element_type=jnp.float32)

    nx = jnp.sqrt(jnp.sum(h * h, axis=0, keepdims=True))
    ny = jnp.sqrt(jnp.sum(y * y, axis=0, keepdims=True))
    ipT = jax.lax.dot_general(y, h, (((0,), (0,)), ((), ())),
                              preferred_element_type=jnp.float32)
    inv_x = 1.0 / (nx + 1e-12)
    inv_y = 1.0 / (ny + 1e-12)
    innerT = ipT * inv_x * inv_y.reshape(_NR, 1)
    sx = (nx * inv_x) ** 2
    sy = (ny * inv_y) ** 2
    distT = 2.0 * innerT - sx - sy.reshape(_NR, 1) + relT_ref[...]

    iota0 = jax.lax.broadcasted_iota(jnp.int32, (_NR, _N), 0)
    d = distT
    rows = []
    for _ in range(_K):
        cur = jnp.max(d, axis=0, keepdims=True)
        first = jnp.min(jnp.where(d >= cur, iota0, _NR), axis=0,
                        keepdims=True)
        rows.append(first)
        d = jnp.where(iota0 == first, -_BIG, d)
    idxmat = jnp.concatenate(rows, axis=0)
    for q in range(_N // _NODES_PER_W):
        idx_ref[0, q] = idxmat[:, q * _NODES_PER_W:(q + 1) * _NODES_PER_W]


def _sc_body(yT_hbm, idx_hbm, out_hbm, yT_v, idx_v, acc_v):
    wid = lax.axis_index("s") * _NC + lax.axis_index("c")
    b = wid // (_N // _NODES_PER_W)
    q = wid % (_N // _NODES_PER_W)

    pltpu.sync_copy(yT_hbm.at[b], yT_v)
    pltpu.sync_copy(idx_hbm.at[b, q], idx_v)

    cu = 8

    def grp_body(g, carry):
        base = g * _L
        ivs = [idx_v[k, pl.ds(base, _L)] for k in range(_K)]

        def c_body(cc, carry2):
            c0 = cc * cu
            for u in range(cu):
                csplat = jnp.full((_L,), 0, jnp.int32) + (c0 + u)
                o = plsc.load_gather(yT_v, [ivs[0], csplat])
                for k in range(1, _K):
                    o = jnp.maximum(o, plsc.load_gather(yT_v, [ivs[k], csplat]))
                acc_v[c0 + u, pl.ds(base, _L)] = o
            return carry2

        return lax.fori_loop(0, _C // cu, c_body, carry)

    lax.fori_loop(0, _NODES_PER_W // _L, grp_body, 0)
    pltpu.sync_copy(acc_v, out_hbm.at[b, q])


def _tc2_body(h_ref, acc_ref, x_ref, mrw_ref, mrb_ref, mrg_ref, mrbeta_ref,
              w2_ref, b2_ref, out_ref):
    h = h_ref[0]
    acc = jnp.concatenate(
        [acc_ref[0, q] for q in range(_N // _NODES_PER_W)], axis=1)
    cat = jnp.concatenate([h, acc - h], axis=0)
    g = jnp.dot(mrw_ref[...], cat,
                preferred_element_type=jnp.float32) + mrb_ref[...]

    rows = (2 * _C) // _GROUPS
    parts = []
    for gi in range(_GROUPS):
        sub = g[gi * rows:(gi + 1) * rows, :]
        m = jnp.mean(sub)
        dsub = sub - m
        v = jnp.mean(dsub * dsub)
        parts.append(dsub * jax.lax.rsqrt(v + 1e-5))
    gn = jnp.concatenate(parts, axis=0) * mrg_ref[...] + mrbeta_ref[...]
    act = jax.nn.gelu(gn)

    out_ref[0] = (jnp.dot(w2_ref[...], act, preferred_element_type=jnp.float32)
                  + b2_ref[...] + x_ref[0])


def kernel(x, fc1_w, fc1_b, fc1_g, fc1_beta, mr_w, mr_b, mr_g, mr_beta,
           fc2_w, fc2_b, fc2_g, fc2_beta):
    x3 = x.reshape(_B, _C, _N)
    w1 = fc1_g[:, None] * fc1_w
    b1 = (fc1_g * fc1_b + fc1_beta)[:, None]
    w2 = fc2_g[:, None] * fc2_w
    b2 = (fc2_g * fc2_b + fc2_beta)[:, None]
    relT = jnp.asarray(_RELT_NP)
    pool = jnp.asarray(_POOL_NP)

    full = lambda shape: pl.BlockSpec(shape, lambda b: (0,) * len(shape))
    batch = lambda shape: pl.BlockSpec((1,) + shape,
                                       lambda b: (b,) + (0,) * len(shape))

    h, yT, idx = pl.pallas_call(
        _tc1_body,
        grid=(_B,),
        in_specs=[
            batch((_C, _N)),
            full((_C, _C)), full((_C, 1)),
            full((_NR, _N)), full((_N, _NR)),
        ],
        out_specs=[batch((_C, _N)), batch((_NR, _C)),
                   batch((_N // _NODES_PER_W, _K, _NODES_PER_W))],
        out_shape=[
            jax.ShapeDtypeStruct((_B, _C, _N), jnp.float32),
            jax.ShapeDtypeStruct((_B, _NR, _C), jnp.float32),
            jax.ShapeDtypeStruct(
                (_B, _N // _NODES_PER_W, _K, _NODES_PER_W), jnp.int32),
        ],
    )(x3, w1, b1, relT, pool)

    mesh = plsc.VectorSubcoreMesh(core_axis_name="c", subcore_axis_name="s")
    acc = pl.kernel(
        _sc_body, mesh=mesh,
        compiler_params=pltpu.CompilerParams(needs_layout_passes=False),
        out_type=jax.ShapeDtypeStruct(
            (_B, _N // _NODES_PER_W, _C, _NODES_PER_W), jnp.float32),
        scratch_types=[
            pltpu.VMEM((_NR, _C), jnp.float32),
            pltpu.VMEM((_K, _NODES_PER_W), jnp.int32),
            pltpu.VMEM((_C, _NODES_PER_W), jnp.float32),
        ],
    )(yT, idx)

    out = pl.pallas_call(
        _tc2_body,
        grid=(_B,),
        in_specs=[
            batch((_C, _N)),
            batch((_N // _NODES_PER_W, _C, _NODES_PER_W)),
            batch((_C, _N)),
            full((2 * _C, 2 * _C)), full((2 * _C, 1)),
            full((2 * _C, 1)), full((2 * _C, 1)),
            full((_C, 2 * _C)), full((_C, 1)),
        ],
        out_specs=batch((_C, _N)),
        out_shape=jax.ShapeDtypeStruct((_B, _C, _N), jnp.float32),
    )(h, acc, x3, mr_w, mr_b[:, None], mr_g[:, None], mr_beta[:, None],
      w2, b2)
    return out.reshape(_B, _C, _H, _W)


# speedup vs baseline: 3.1805x; 3.1733x over previous
import functools
import math

import jax
import jax.numpy as jnp
import numpy as np
from jax import lax
from jax.experimental import pallas as pl
from jax.experimental.pallas import tpu as pltpu
from jax.experimental.pallas import tpu_sc as plsc

_B, _C, _H, _W = 8, 96, 56, 56
_K = 9
_HR, _WR = 7, 7
_N = _H * _W
_NR = _HR * _WR
_GROUPS = 4
_BIG = 3.0e38

_NC, _NS, _L = 2, 16, 16
_NW = _NC * _NS
_NODES_PER_W = (_B * _N) // _NW


def _pos_embed_np(c, h, w):
    d = c // 2
    pe = np.zeros((c, h, w), dtype=np.float32)
    div = np.exp(np.arange(0.0, d, 2) * -(math.log(10000.0) / d))
    pos_w = np.arange(0.0, w)[:, None]
    pos_h = np.arange(0.0, h)[:, None]
    pe[0:d:2, :, :] = np.sin(pos_w * div).T[:, None, :]
    pe[1:d:2, :, :] = np.cos(pos_w * div).T[:, None, :]
    pe[d::2, :, :] = np.sin(pos_h * div).T[:, :, None]
    pe[d + 1::2, :, :] = np.cos(pos_h * div).T[:, :, None]
    return pe


def _constants():
    pos = _pos_embed_np(_C, _H, _W)
    pos_red = pos.reshape(_C, _HR, _H // _HR, _WR, _W // _WR).mean(axis=(2, 4))
    rel = 2.0 * (pos.reshape(_C, -1).T @ pos_red.reshape(_C, -1)) / _C
    relT = np.ascontiguousarray(rel.T).astype(np.float32)
    hh = np.arange(_H)[:, None]
    ww = np.arange(_W)[None, :]
    blk = (hh // (_H // _HR)) * _WR + (ww // (_W // _WR))
    pool = (blk.reshape(_N, 1) == np.arange(_NR)[None, :]).astype(np.float32) / 64.0
    return relT, pool


_RELT_NP, _POOL_NP = _constants()


def _tc1_body(x_ref, w1_ref, b1_ref, relT_ref, pool_ref,
              h_ref, yT_ref, idx_ref):
    x = x_ref[0]
    h = jnp.dot(w1_ref[...], x, preferred_element_type=jnp.float32) + b1_ref[...]
    h_ref[0] = h

    y = jnp.dot(h, pool_ref[...], preferred_element_type=jnp.float32)
    yT_ref[0] = y

    nx = jnp.sqrt(jnp.sum(h * h, axis=0, keepdims=True))
    ny = jnp.sqrt(jnp.sum(y * y, axis=0, keepdims=True))
    ipT = jax.lax.dot_general(y, h, (((0,), (0,)), ((), ())),
                              preferred_element_type=jnp.float32)
    inv_x = 1.0 / (nx + 1e-12)
    inv_y = 1.0 / (ny + 1e-12)
    innerT = ipT * inv_x * inv_y.reshape(_NR, 1)
    sx = (nx * inv_x) ** 2
    sy = (ny * inv_y) ** 2
    distT = 2.0 * innerT - sx - sy.reshape(_NR, 1) + relT_ref[...]

    iota0 = jax.lax.broadcasted_iota(jnp.int32, (_NR, _N), 0)
    d = distT
    rows = []
    for _ in range(_K):
        cur = jnp.max(d, axis=0, keepdims=True)
        first = jnp.min(jnp.where(d >= cur, iota0, _NR), axis=0,
                        keepdims=True)
        rows.append(first)
        d = jnp.where(iota0 == first, -_BIG, d)
    idxmat = jnp.concatenate(rows, axis=0)
    for q in range(_N // _NODES_PER_W):
        idx_ref[0, q] = idxmat[:, q * _NODES_PER_W:(q + 1) * _NODES_PER_W]


def _sc_body(yT_hbm, idx_hbm, out_hbm, yT_v, idx_v, acc_v):
    wid = lax.axis_index("s") * _NC + lax.axis_index("c")
    b = wid // (_N // _NODES_PER_W)
    q = wid % (_N // _NODES_PER_W)

    pltpu.sync_copy(yT_hbm.at[b], yT_v)
    pltpu.sync_copy(idx_hbm.at[b, q], idx_v)

    cu = 8

    def grp_body(g, carry):
        base = g * _L
        ivs = [idx_v[k, pl.ds(base, _L)] for k in range(_K)]

        def c_body(cc, carry2):
            c0 = cc * cu
            for u in range(cu):
                csplat = jnp.full((_L,), 0, jnp.int32) + (c0 + u)
                o = plsc.load_gather(yT_v, [csplat, ivs[0]])
                for k in range(1, _K):
                    o = jnp.maximum(o, plsc.load_gather(yT_v, [csplat, ivs[k]]))
                acc_v[c0 + u, pl.ds(base, _L)] = o
            return carry2

        return lax.fori_loop(0, _C // cu, c_body, carry)

    lax.fori_loop(0, _NODES_PER_W // _L, grp_body, 0)
    pltpu.sync_copy(acc_v, out_hbm.at[b, q])


def _tc2_body(h_ref, acc_ref, x_ref, mrw_ref, mrb_ref, mrg_ref, mrbeta_ref,
              w2_ref, b2_ref, out_ref):
    h = h_ref[0]
    acc = jnp.concatenate(
        [acc_ref[0, q] for q in range(_N // _NODES_PER_W)], axis=1)
    cat = jnp.concatenate([h, acc - h], axis=0)
    g = jnp.dot(mrw_ref[...], cat,
                preferred_element_type=jnp.float32) + mrb_ref[...]

    rows = (2 * _C) // _GROUPS
    parts = []
    for gi in range(_GROUPS):
        sub = g[gi * rows:(gi + 1) * rows, :]
        m = jnp.mean(sub)
        dsub = sub - m
        v = jnp.mean(dsub * dsub)
        parts.append(dsub * jax.lax.rsqrt(v + 1e-5))
    gn = jnp.concatenate(parts, axis=0) * mrg_ref[...] + mrbeta_ref[...]
    act = jax.nn.gelu(gn)

    out_ref[0] = (jnp.dot(w2_ref[...], act, preferred_element_type=jnp.float32)
                  + b2_ref[...] + x_ref[0])


def kernel(x, fc1_w, fc1_b, fc1_g, fc1_beta, mr_w, mr_b, mr_g, mr_beta,
           fc2_w, fc2_b, fc2_g, fc2_beta):
    x3 = x.reshape(_B, _C, _N)
    w1 = fc1_g[:, None] * fc1_w
    b1 = (fc1_g * fc1_b + fc1_beta)[:, None]
    w2 = fc2_g[:, None] * fc2_w
    b2 = (fc2_g * fc2_b + fc2_beta)[:, None]
    relT = jnp.asarray(_RELT_NP)
    pool = jnp.asarray(_POOL_NP)

    full = lambda shape: pl.BlockSpec(shape, lambda b: (0,) * len(shape))
    batch = lambda shape: pl.BlockSpec((1,) + shape,
                                       lambda b: (b,) + (0,) * len(shape))

    h, yT, idx = pl.pallas_call(
        _tc1_body,
        grid=(_B,),
        in_specs=[
            batch((_C, _N)),
            full((_C, _C)), full((_C, 1)),
            full((_NR, _N)), full((_N, _NR)),
        ],
        out_specs=[batch((_C, _N)), batch((_C, _NR)),
                   batch((_N // _NODES_PER_W, _K, _NODES_PER_W))],
        out_shape=[
            jax.ShapeDtypeStruct((_B, _C, _N), jnp.float32),
            jax.ShapeDtypeStruct((_B, _C, _NR), jnp.float32),
            jax.ShapeDtypeStruct(
                (_B, _N // _NODES_PER_W, _K, _NODES_PER_W), jnp.int32),
        ],
    )(x3, w1, b1, relT, pool)

    mesh = plsc.VectorSubcoreMesh(core_axis_name="c", subcore_axis_name="s")
    acc = pl.kernel(
        _sc_body, mesh=mesh,
        compiler_params=pltpu.CompilerParams(needs_layout_passes=False),
        out_type=jax.ShapeDtypeStruct(
            (_B, _N // _NODES_PER_W, _C, _NODES_PER_W), jnp.float32),
        scratch_types=[
            pltpu.VMEM((_C, _NR), jnp.float32),
            pltpu.VMEM((_K, _NODES_PER_W), jnp.int32),
            pltpu.VMEM((_C, _NODES_PER_W), jnp.float32),
        ],
    )(yT, idx)

    out = pl.pallas_call(
        _tc2_body,
        grid=(_B,),
        in_specs=[
            batch((_C, _N)),
            batch((_N // _NODES_PER_W, _C, _NODES_PER_W)),
            batch((_C, _N)),
            full((2 * _C, 2 * _C)), full((2 * _C, 1)),
            full((2 * _C, 1)), full((2 * _C, 1)),
            full((_C, 2 * _C)), full((_C, 1)),
        ],
        out_specs=batch((_C, _N)),
        out_shape=jax.ShapeDtypeStruct((_B, _C, _N), jnp.float32),
    )(h, acc, x3, mr_w, mr_b[:, None], mr_g[:, None], mr_beta[:, None],
      w2, b2)
    return out.reshape(_B, _C, _H, _W)
